# baseline (speedup 1.0000x reference)
import functools

import jax
import jax.numpy as jnp
from jax import lax
from jax.experimental import pallas as pl
from jax.experimental.pallas import tpu as pltpu
from jax.experimental.pallas import tpu_sc as plsc

N = 10000
E = 320000
D = 128
H = 128

NC = 2
NS = 16
NW = NC * NS
EPW = E // NW
CHUNK_ROWS = 80
NCHUNK = EPW // CHUNK_ROWS
ACHUNK = 80
ANCHUNK = EPW // ACHUNK
NPAD = 10240
ROWS_PER_TILE = NPAD // NS
ZROWS = 128


def _sc_mesh():
    return plsc.VectorSubcoreMesh(
        core_axis_name="c", subcore_axis_name="s", num_cores=NC, num_subcores=NS
    )


def _sc_degree_body(dst_hbm, degp_hbm, didx_v, ones_v, zbuf_v, acc_s):
    c = lax.axis_index("c")
    s = lax.axis_index("s")
    wid = c * NS + s

    def init_ones(i, carry):
        ones_v[pl.ds(i * 16, 16)] = jnp.ones((16,), jnp.float32)
        return carry

    lax.fori_loop(0, CHUNK_ROWS // 16, init_ones, 0)

    def init_zero(i, carry):
        zbuf_v[pl.ds(i * 16, 16)] = jnp.zeros((16,), jnp.float32)
        return carry

    lax.fori_loop(0, N // 16, init_zero, 0)

    @pl.when(s == 0)
    def _():
        pltpu.sync_copy(zbuf_v, acc_s)

    pltpu.sync_copy(dst_hbm.at[wid], didx_v)
    plsc.subcore_barrier()

    def body(j, carry):
        pltpu.sync_copy(ones_v, acc_s.at[didx_v.at[j]], add=True)
        return carry

    lax.fori_loop(0, NCHUNK, body, 0)
    plsc.subcore_barrier()

    @pl.when(s == 0)
    def _():
        pltpu.sync_copy(acc_s, degp_hbm.at[c])


def _sc_degree(dst):
    return pl.kernel(
        _sc_degree_body,
        out_type=jax.ShapeDtypeStruct((NC, N), jnp.float32),
        mesh=_sc_mesh(),
        scratch_types=[
            pltpu.VMEM((NCHUNK, CHUNK_ROWS), jnp.int32),
            pltpu.VMEM((CHUNK_ROWS,), jnp.float32),
            pltpu.VMEM((N,), jnp.float32),
            pltpu.VMEM_SHARED((N,), jnp.float32),
        ],
    )(dst)


def _sc_agg_body(feat_hbm, pidx_hbm, out_hbm,
                 pidx_v, si_v, di_v, gbuf_v, acc_s, sem_g, sem_s):
    c = lax.axis_index("c")
    s = lax.axis_index("s")
    wid = c * NS + s

    def init_zero(i, carry):
        gbuf_v[0, i // 8, pl.ds((i % 8) * 16, 16)] = jnp.zeros(
            (16,), jnp.float32)
        return carry

    lax.fori_loop(0, ACHUNK * 8, init_zero, 0)

    for k in range(ROWS_PER_TILE // ACHUNK):
        pltpu.sync_copy(
            gbuf_v.at[0],
            acc_s.at[pl.ds(s * ROWS_PER_TILE + k * ACHUNK, ACHUNK)]
        )

    pltpu.sync_copy(pidx_hbm.at[wid], pidx_v)
    plsc.subcore_barrier()

    def unpack(j, b):
        for k in range(ACHUNK // 16):
            pk = pidx_v[j, pl.ds(k * 16, 16)]
            si_v[b, pl.ds(k * 16, 16)] = pk & 0xFFFF
            di_v[b, pl.ds(k * 16, 16)] = lax.shift_right_logical(pk, 16)

    def gstart(b):
        pltpu.async_copy(feat_hbm.at[si_v.at[b]], gbuf_v.at[b], sem_g.at[b])

    def gwait(b):
        pltpu.make_async_copy(
            feat_hbm.at[si_v.at[b]], gbuf_v.at[b], sem_g.at[b]).wait()

    def sstart(b):
        pltpu.async_copy(gbuf_v.at[b], acc_s.at[di_v.at[b]],
                         sem_s.at[b], add=True)

    def swait(b):
        pltpu.make_async_copy(gbuf_v.at[b], acc_s.at[di_v.at[b]],
                              sem_s.at[b]).wait()

    unpack(0, 0)
    gstart(0)
    unpack(1, 1)
    gstart(1)
    gwait(0)
    sstart(0)

    def pair(q, carry):
        j = 1 + 2 * q
        gwait(1)
        swait(0)
        sstart(1)
        unpack(j + 1, 0)
        gstart(0)
        gwait(0)
        swait(1)
        sstart(0)
        unpack(j + 2, 1)
        gstart(1)
        return carry

    lax.fori_loop(0, (ANCHUNK - 3) // 2, pair, 0)

    gwait(1)
    swait(0)
    sstart(1)
    unpack(ANCHUNK - 1, 0)
    gstart(0)
    gwait(0)
    swait(1)
    sstart(0)
    swait(0)
    plsc.subcore_barrier()

    for k in range(ROWS_PER_TILE // ZROWS):
        sl = pl.ds(s * ROWS_PER_TILE + k * ZROWS, ZROWS)
        pltpu.sync_copy(acc_s.at[sl], out_hbm.at[c, sl])


_AGG_SCRATCH = [
    pltpu.VMEM((ANCHUNK, ACHUNK), jnp.int32),
    pltpu.VMEM((2, ACHUNK), jnp.int32),
    pltpu.VMEM((2, ACHUNK), jnp.int32),
    pltpu.VMEM((2, ACHUNK, D), jnp.float32),
    pltpu.VMEM_SHARED((NPAD, D), jnp.float32),
    pltpu.SemaphoreType.DMA((2,)),
    pltpu.SemaphoreType.DMA((2,)),
]


def _sc_agg(feat, pidx):
    return pl.kernel(
        _sc_agg_body,
        out_type=jax.ShapeDtypeStruct((NC, NPAD, D), jnp.float32),
        mesh=_sc_mesh(),
        scratch_types=_AGG_SCRATCH,
    )(feat, pidx)


def _tc_lin_body(x_ref, w_ref, b_ref, d0_ref, d1_ref, hn_ref, norm_ref):
    deg = d0_ref[...] + d1_ref[...]
    norm = lax.rsqrt(jnp.maximum(deg, 1.0))
    h = jnp.dot(x_ref[...], w_ref[...], preferred_element_type=jnp.float32)
    h = h + b_ref[...]
    hn_ref[...] = h * norm
    norm_ref[...] = norm


def _tc_lin(x, w, b, d0, d1):
    return pl.pallas_call(
        _tc_lin_body,
        out_shape=(
            jax.ShapeDtypeStruct((N, D), jnp.float32),
            jax.ShapeDtypeStruct((N, 1), jnp.float32),
        ),
    )(x, w, b, d0, d1)


def _tc_layer_body(p0_ref, p1_ref, norm_ref, w_ref, b_ref, a_ref, out_ref,
                   *, scale_out):
    norm = norm_ref[...]
    agg = (p0_ref[...][:N] + p1_ref[...][:N]) * norm
    t = jnp.dot(agg, w_ref[...], preferred_element_type=jnp.float32)
    t = t + b_ref[...]
    h = jnp.where(t >= 0, t, a_ref[...] * t)
    out_ref[...] = h * norm if scale_out else h


def _tc_layer(p0, p1, norm, w, b, a, scale_out):
    body = functools.partial(_tc_layer_body, scale_out=scale_out)
    return pl.pallas_call(
        body,
        out_shape=jax.ShapeDtypeStruct((N, H), jnp.float32),
    )(p0, p1, norm, w, b, a)


def kernel(x, edge_index, W_lin, b_lin, W_g1, b_g1, W_g2, b_g2, a1, a2):
    dst = edge_index[1].reshape(NW, NCHUNK, CHUNK_ROWS)
    packed = (edge_index[0] | (edge_index[1] << 16)).reshape(
        NW, ANCHUNK, ACHUNK)

    degp = _sc_degree(dst)
    hn, norm = _tc_lin(
        x, W_lin, b_lin.reshape(1, D),
        degp[0].reshape(N, 1), degp[1].reshape(N, 1),
    )
    p = _sc_agg(hn, packed)
    hn1 = _tc_layer(p[0], p[1], norm, W_g1, b_g1.reshape(1, H),
                    a1.reshape(1, 1), scale_out=True)
    q = _sc_agg(hn1, packed)
    h2 = _tc_layer(q[0], q[1], norm, W_g2, b_g2.reshape(1, H),
                   a2.reshape(1, 1), scale_out=False)
    return h2


# speedup vs baseline: 1.3868x; 1.3868x over previous
import functools

import jax
import jax.numpy as jnp
from jax import lax
from jax.experimental import pallas as pl
from jax.experimental.pallas import tpu as pltpu
from jax.experimental.pallas import tpu_sc as plsc

N = 10000
E = 320000
D = 128
H = 128

NC = 2
NS = 16
NW = NC * NS
EPW = E // NW
CHUNK_ROWS = 80
NCHUNK = EPW // CHUNK_ROWS
ACHUNK = 80
ANCHUNK = EPW // ACHUNK
NPAD = 10240
ROWS_PER_TILE = NPAD // NS
ZROWS = 128


def _sc_mesh():
    return plsc.VectorSubcoreMesh(
        core_axis_name="c", subcore_axis_name="s", num_cores=NC, num_subcores=NS
    )


def _sc_degree_body(dst_hbm, degp_hbm, didx_v, ones_v, zbuf_v, acc_s):
    c = lax.axis_index("c")
    s = lax.axis_index("s")
    wid = c * NS + s

    def init_ones(i, carry):
        ones_v[pl.ds(i * 16, 16)] = jnp.ones((16,), jnp.float32)
        return carry

    lax.fori_loop(0, CHUNK_ROWS // 16, init_ones, 0)

    def init_zero(i, carry):
        zbuf_v[pl.ds(i * 16, 16)] = jnp.zeros((16,), jnp.float32)
        return carry

    lax.fori_loop(0, N // 16, init_zero, 0)

    @pl.when(s == 0)
    def _():
        pltpu.sync_copy(zbuf_v, acc_s)

    pltpu.sync_copy(dst_hbm.at[wid], didx_v)
    plsc.subcore_barrier()

    def body(j, carry):
        pltpu.sync_copy(ones_v, acc_s.at[didx_v.at[j]], add=True)
        return carry

    lax.fori_loop(0, NCHUNK, body, 0)
    plsc.subcore_barrier()

    @pl.when(s == 0)
    def _():
        pltpu.sync_copy(acc_s, degp_hbm.at[c])


def _sc_degree(dst):
    return pl.kernel(
        _sc_degree_body,
        out_type=jax.ShapeDtypeStruct((NC, N), jnp.float32),
        mesh=_sc_mesh(),
        scratch_types=[
            pltpu.VMEM((NCHUNK, CHUNK_ROWS), jnp.int32),
            pltpu.VMEM((CHUNK_ROWS,), jnp.float32),
            pltpu.VMEM((N,), jnp.float32),
            pltpu.VMEM_SHARED((N,), jnp.float32),
        ],
    )(dst)


def _sc_agg_body(feat_hbm, pidx_hbm, out_hbm,
                 pidx_v, si_v, di_v, gbuf_v, acc_s, sem_g, sem_s):
    c = lax.axis_index("c")
    s = lax.axis_index("s")
    wid = c * NS + s

    def init_zero(i, carry):
        gbuf_v[0, i // 8, pl.ds((i % 8) * 16, 16)] = jnp.zeros(
            (16,), jnp.float32)
        return carry

    lax.fori_loop(0, ACHUNK * 8, init_zero, 0)

    for k in range(ROWS_PER_TILE // ACHUNK):
        pltpu.sync_copy(
            gbuf_v.at[0],
            acc_s.at[pl.ds(s * ROWS_PER_TILE + k * ACHUNK, ACHUNK)]
        )

    pltpu.sync_copy(pidx_hbm.at[wid], pidx_v)
    plsc.subcore_barrier()

    def unpack(j, b):
        for k in range(ACHUNK // 16):
            pk = pidx_v[j, pl.ds(k * 16, 16)]
            si_v[b, pl.ds(k * 16, 16)] = pk & 0xFFFF
            di_v[b, pl.ds(k * 16, 16)] = lax.shift_right_logical(pk, 16)

    def gstart(b):
        pltpu.async_copy(feat_hbm.at[si_v.at[b]], gbuf_v.at[b], sem_g.at[b])

    def gwait(b):
        pltpu.make_async_copy(
            feat_hbm.at[si_v.at[b]], gbuf_v.at[b], sem_g.at[b]).wait()

    def sstart(b):
        pltpu.async_copy(gbuf_v.at[b], acc_s.at[di_v.at[b]],
                         sem_s.at[b], add=True)

    def swait(b):
        pltpu.make_async_copy(gbuf_v.at[b], acc_s.at[di_v.at[b]],
                              sem_s.at[b]).wait()

    unpack(0, 0)
    gstart(0)
    unpack(1, 1)
    gstart(1)
    gwait(0)
    sstart(0)
    unpack(2, 2)
    gstart(2)

    def step(j, bg, bp):
        gwait(bg)
        swait(bp)
        sstart(bg)
        unpack(j + 2, bp)
        gstart(bp)

    def triple(q, carry):
        j = 1 + 3 * q
        step(j, 1, 0)
        step(j + 1, 2, 1)
        step(j + 2, 0, 2)
        return carry

    lax.fori_loop(0, (ANCHUNK - 5) // 3, triple, 0)

    step(ANCHUNK - 4, 1, 0)
    step(ANCHUNK - 3, 2, 1)
    gwait(0)
    swait(2)
    sstart(0)
    gwait(1)
    swait(0)
    sstart(1)
    swait(1)
    plsc.subcore_barrier()

    for k in range(ROWS_PER_TILE // ZROWS):
        sl = pl.ds(s * ROWS_PER_TILE + k * ZROWS, ZROWS)
        pltpu.sync_copy(acc_s.at[sl], out_hbm.at[c, sl])


_AGG_SCRATCH = [
    pltpu.VMEM((ANCHUNK, ACHUNK), jnp.int32),
    pltpu.VMEM((3, ACHUNK), jnp.int32),
    pltpu.VMEM((3, ACHUNK), jnp.int32),
    pltpu.VMEM((3, ACHUNK, D), jnp.float32),
    pltpu.VMEM_SHARED((NPAD, D), jnp.float32),
    pltpu.SemaphoreType.DMA((3,)),
    pltpu.SemaphoreType.DMA((3,)),
]


def _sc_agg(feat, pidx):
    return pl.kernel(
        _sc_agg_body,
        out_type=jax.ShapeDtypeStruct((NC, NPAD, D), jnp.float32),
        mesh=_sc_mesh(),
        scratch_types=_AGG_SCRATCH,
    )(feat, pidx)


def _tc_lin_body(x_ref, w_ref, b_ref, d0_ref, d1_ref, hn_ref, norm_ref):
    deg = d0_ref[...] + d1_ref[...]
    norm = lax.rsqrt(jnp.maximum(deg, 1.0))
    h = jnp.dot(x_ref[...], w_ref[...], preferred_element_type=jnp.float32)
    h = h + b_ref[...]
    hn_ref[...] = h * norm
    norm_ref[...] = norm


def _tc_lin(x, w, b, d0, d1):
    return pl.pallas_call(
        _tc_lin_body,
        out_shape=(
            jax.ShapeDtypeStruct((N, D), jnp.float32),
            jax.ShapeDtypeStruct((N, 1), jnp.float32),
        ),
    )(x, w, b, d0, d1)


def _tc_layer_body(p0_ref, p1_ref, norm_ref, w_ref, b_ref, a_ref, out_ref,
                   *, scale_out):
    norm = norm_ref[...]
    agg = (p0_ref[...][:N] + p1_ref[...][:N]) * norm
    t = jnp.dot(agg, w_ref[...], preferred_element_type=jnp.float32)
    t = t + b_ref[...]
    h = jnp.where(t >= 0, t, a_ref[...] * t)
    out_ref[...] = h * norm if scale_out else h


def _tc_layer(p0, p1, norm, w, b, a, scale_out):
    body = functools.partial(_tc_layer_body, scale_out=scale_out)
    return pl.pallas_call(
        body,
        out_shape=jax.ShapeDtypeStruct((N, H), jnp.float32),
    )(p0, p1, norm, w, b, a)


def kernel(x, edge_index, W_lin, b_lin, W_g1, b_g1, W_g2, b_g2, a1, a2):
    dst = edge_index[1].reshape(NW, NCHUNK, CHUNK_ROWS)
    packed = (edge_index[0] | (edge_index[1] << 16)).reshape(
        NW, ANCHUNK, ACHUNK)

    degp = _sc_degree(dst)
    hn, norm = _tc_lin(
        x, W_lin, b_lin.reshape(1, D),
        degp[0].reshape(N, 1), degp[1].reshape(N, 1),
    )
    p = _sc_agg(hn, packed)
    hn1 = _tc_layer(p[0], p[1], norm, W_g1, b_g1.reshape(1, H),
                    a1.reshape(1, 1), scale_out=True)
    q = _sc_agg(hn1, packed)
    h2 = _tc_layer(q[0], q[1], norm, W_g2, b_g2.reshape(1, H),
                   a2.reshape(1, 1), scale_out=False)
    return h2
